# baseline (speedup 1.0000x reference)
import functools

import jax
import jax.numpy as jnp
from jax import lax
from jax.experimental import pallas as pl
from jax.experimental.pallas import tpu as pltpu
from jax.experimental.pallas import tpu_sc as plsc

B, S, H = 4, 2048, 4096
N = B * S
NC, NS, L = 2, 16, 16
NW = NC * NS
TOK = N // NW
T = 8
NCHUNK = TOK // T
EPS = 1e-5


def _allsum16(x):
    iota = lax.iota(jnp.int32, L)
    dnums = lax.GatherDimensionNumbers(
        offset_dims=(), collapsed_slice_dims=(0,), start_index_map=(0,))
    for k in (1, 2, 4, 8):
        perm = lax.bitwise_and(iota + jnp.int32(k), jnp.int32(L - 1))
        x = x + lax.gather(
            x, perm[:, None], dnums, slice_sizes=(1,),
            unique_indices=True, indices_are_sorted=False,
            mode=lax.GatherScatterMode.PROMISE_IN_BOUNDS)
    return x


def _rsqrt16(v):
    i = lax.bitcast_convert_type(v, jnp.int32)
    i = jnp.int32(0x5F3759DF) - lax.shift_right_arithmetic(i, jnp.int32(1))
    y = lax.bitcast_convert_type(i, jnp.float32)
    half = jnp.float32(0.5) * v
    for _ in range(3):
        y = y * (jnp.float32(1.5) - half * y * y)
    return y


def _sc_body(emb_hbm, idx_hbm, table_hbm, gamma_hbm, beta_hbm, out_hbm,
             idx_v, emb_v, row_v, gamma_v, beta_v, sem):
    wid = lax.axis_index("s") * NC + lax.axis_index("c")
    base = wid * TOK

    pltpu.sync_copy(idx_hbm.at[pl.ds(base, TOK)], idx_v)
    pltpu.sync_copy(gamma_hbm, gamma_v)
    pltpu.sync_copy(beta_hbm, beta_v)

    def _addone(i, c):
        sl = pl.ds(i * L, L)
        idx_v[sl] = idx_v[sl] + jnp.int32(1)
        return c
    lax.fori_loop(0, TOK // L, _addone, 0)

    inv_h = jnp.float32(1.0 / H)

    def _chunk(ct, c):
        t0 = base + ct * T
        pltpu.async_copy(emb_hbm.at[pl.ds(t0, T)], emb_v, sem).wait()
        pltpu.async_copy(table_hbm.at[idx_v.at[pl.ds(ct * T, T)]],
                         row_v, sem).wait()

        def _token(t, c2):
            zero = jnp.zeros((L,), jnp.float32)

            def _p1(j, carry):
                s, s2 = carry
                sl = pl.ds(j * L, L)
                x = emb_v[t, sl] + row_v[t, sl]
                emb_v[t, sl] = x
                return (s + x, s2 + x * x)

            s, s2 = lax.fori_loop(0, H // L, _p1, (zero, zero))
            mean = _allsum16(s) * inv_h
            var = _allsum16(s2) * inv_h - mean * mean
            rstd = _rsqrt16(var + jnp.float32(EPS))

            def _p2(j, c3):
                sl = pl.ds(j * L, L)
                x = emb_v[t, sl]
                emb_v[t, sl] = (x - mean) * rstd * gamma_v[sl] + beta_v[sl]
                return c3

            lax.fori_loop(0, H // L, _p2, 0)
            return c2

        lax.fori_loop(0, T, _token, 0)
        pltpu.async_copy(emb_v, out_hbm.at[pl.ds(t0, T)], sem).wait()
        return c

    lax.fori_loop(0, NCHUNK, _chunk, 0)


@jax.jit
def _run(emb2d, idx1d, table, gamma, beta):
    mesh = plsc.VectorSubcoreMesh(core_axis_name="c", subcore_axis_name="s",
                                  num_cores=NC, num_subcores=NS)
    f = pl.kernel(
        _sc_body,
        out_type=jax.ShapeDtypeStruct((N, H), jnp.float32),
        mesh=mesh,
        scratch_types=[
            pltpu.VMEM((TOK,), jnp.int32),
            pltpu.VMEM((T, H), jnp.float32),
            pltpu.VMEM((T, H), jnp.float32),
            pltpu.VMEM((H,), jnp.float32),
            pltpu.VMEM((H,), jnp.float32),
            pltpu.SemaphoreType.DMA,
        ],
    )
    return f(emb2d, idx1d, table, gamma, beta)


def kernel(embedded_input_ids, position_ids, pos_table, ln_gamma, ln_beta):
    emb2d = embedded_input_ids.reshape(N, H)
    idx1d = position_ids.astype(jnp.int32).reshape(N)
    out = _run(emb2d, idx1d, pos_table, ln_gamma, ln_beta)
    return out.reshape(B, S, H)


# speedup vs baseline: 1.8397x; 1.8397x over previous
import jax
import jax.numpy as jnp
from jax import lax
from jax.experimental import pallas as pl
from jax.experimental.pallas import tpu as pltpu
from jax.experimental.pallas import tpu_sc as plsc

B, S, H = 4, 2048, 4096
N = B * S
NC, NS, L = 2, 16, 16
NW = NC * NS
TOK = N // NW
T = 4
NCHUNK = TOK // T
NBE = 3
NBR = 2
NU = 16
IST = 8
EPS = 1e-5


def _allsum16(x):
    iota = lax.iota(jnp.int32, L)
    dnums = lax.GatherDimensionNumbers(
        offset_dims=(), collapsed_slice_dims=(0,), start_index_map=(0,))
    for k in (1, 2, 4, 8):
        perm = lax.bitwise_and(iota + jnp.int32(k), jnp.int32(L - 1))
        x = x + lax.gather(
            x, perm[:, None], dnums, slice_sizes=(1,),
            unique_indices=True, indices_are_sorted=False,
            mode=lax.GatherScatterMode.PROMISE_IN_BOUNDS)
    return x


def _rsqrt16(v):
    i = lax.bitcast_convert_type(v, jnp.int32)
    i = jnp.int32(0x5F3759DF) - lax.shift_right_arithmetic(i, jnp.int32(1))
    y = lax.bitcast_convert_type(i, jnp.float32)
    half = jnp.float32(0.5) * v
    for _ in range(3):
        y = y * (jnp.float32(1.5) - half * y * y)
    return y


def _sc_body(emb_hbm, idx_hbm, table_hbm, gamma_hbm, beta_hbm, out_hbm,
             idx_v, bufe_v, bufr_v, gamma_v, beta_v,
             sem_emb, sem_gad, sem_out):
    wid = lax.axis_index("s") * NC + lax.axis_index("c")
    base = wid * TOK

    pltpu.sync_copy(idx_hbm.at[pl.ds(wid * (NCHUNK * IST), NCHUNK * IST)],
                    idx_v)
    pltpu.sync_copy(gamma_hbm, gamma_v)
    pltpu.sync_copy(beta_hbm, beta_v)

    def _addone(i, c):
        sl = pl.ds(i * L, L)
        idx_v[sl] = idx_v[sl] + jnp.int32(1)
        return c
    lax.fori_loop(0, NCHUNK * IST // L, _addone, 0)

    inv_h = jnp.float32(1.0 / H)
    zero = jnp.zeros((L,), jnp.float32)

    def emb_start(ct):
        pltpu.async_copy(emb_hbm.at[pl.ds(base + ct * T, T)],
                         bufe_v.at[lax.rem(ct, NBE)], sem_emb)

    def emb_wait():
        pltpu.make_async_copy(emb_hbm.at[pl.ds(base, T)],
                              bufe_v.at[0], sem_emb).wait()

    def gad_start(ct):
        pltpu.async_copy(table_hbm.at[idx_v.at[pl.ds(ct * IST, T)]],
                         bufr_v.at[lax.rem(ct, NBR)], sem_gad)

    def gad_wait():
        pltpu.make_async_copy(emb_hbm.at[pl.ds(base, T)],
                              bufr_v.at[0], sem_gad).wait()

    def out_start(ct):
        pltpu.async_copy(bufe_v.at[lax.rem(ct, NBE)],
                         out_hbm.at[pl.ds(base + ct * T, T)], sem_out)

    def out_wait():
        pltpu.make_async_copy(bufe_v.at[0],
                              out_hbm.at[pl.ds(base, T)], sem_out).wait()

    def compute(ct):
        be = lax.rem(ct, NBE)
        br = lax.rem(ct, NBR)
        aa = []
        cc = []
        for t in range(T):
            def _p1(j, carry):
                s = list(carry[:4])
                q = list(carry[4:])
                off = j * (NU * L)
                for k in range(NU):
                    sl = pl.ds(off + k * L, L)
                    x = bufe_v[be, t, sl] + bufr_v[br, t, sl]
                    bufe_v[be, t, sl] = x
                    s[k % 4] = s[k % 4] + x
                    q[k % 4] = q[k % 4] + x * x
                return (*s, *q)

            r = lax.fori_loop(0, H // (NU * L), _p1, (zero,) * 8)
            s = (r[0] + r[1]) + (r[2] + r[3])
            q = (r[4] + r[5]) + (r[6] + r[7])
            mean = _allsum16(s) * inv_h
            var = _allsum16(q) * inv_h - mean * mean
            rstd = _rsqrt16(var + jnp.float32(EPS))
            aa.append(rstd)
            cc.append(-mean * rstd)

        def _p2(j, c):
            sl = pl.ds(j * L, L)
            g = gamma_v[sl]
            bt = beta_v[sl]
            for t in range(T):
                x = bufe_v[be, t, sl]
                bufe_v[be, t, sl] = (x * aa[t] + cc[t]) * g + bt
            return c

        lax.fori_loop(0, H // L, _p2, 0)

    emb_start(0)
    gad_start(0)
    emb_start(1)
    gad_start(1)

    def _chunk(ct, c):
        @pl.when(ct + 2 < NCHUNK)
        def _():
            @pl.when(ct >= 1)
            def _():
                out_wait()
            emb_start(ct + 2)

        emb_wait()
        gad_wait()
        compute(ct)
        out_start(ct)

        @pl.when(ct + 2 < NCHUNK)
        def _():
            gad_start(ct + 2)
        return c

    lax.fori_loop(0, NCHUNK, _chunk, 0)
    out_wait()
    out_wait()
    out_wait()


@jax.jit
def _run(emb2d, idx1d, table, gamma, beta):
    mesh = plsc.VectorSubcoreMesh(core_axis_name="c", subcore_axis_name="s",
                                  num_cores=NC, num_subcores=NS)
    f = pl.kernel(
        _sc_body,
        out_type=jax.ShapeDtypeStruct((N, H), jnp.float32),
        mesh=mesh,
        scratch_types=[
            pltpu.VMEM((TOK // T * IST,), jnp.int32),
            pltpu.VMEM((NBE, T, H), jnp.float32),
            pltpu.VMEM((NBR, T, H), jnp.float32),
            pltpu.VMEM((H,), jnp.float32),
            pltpu.VMEM((H,), jnp.float32),
            pltpu.SemaphoreType.DMA,
            pltpu.SemaphoreType.DMA,
            pltpu.SemaphoreType.DMA,
        ],
    )
    return f(emb2d, idx1d, table, gamma, beta)


def kernel(embedded_input_ids, position_ids, pos_table, ln_gamma, ln_beta):
    emb2d = embedded_input_ids.reshape(N, H)
    idx1d = position_ids.astype(jnp.int32).reshape(N)
    idxp = (jnp.zeros((N // T, IST), jnp.int32)
            .at[:, :T].set(idx1d.reshape(N // T, T)).reshape(-1))
    out = _run(emb2d, idxp, pos_table, ln_gamma, ln_beta)
    return out.reshape(B, S, H)


# speedup vs baseline: 2.4292x; 1.3204x over previous
import jax
import jax.numpy as jnp
from jax import lax
from jax.experimental import pallas as pl
from jax.experimental.pallas import tpu as pltpu
from jax.experimental.pallas import tpu_sc as plsc

B, S, H = 4, 2048, 4096
N = B * S
NC, NS, L = 2, 16, 16
NW = NC * NS
TOK = N // NW
T = 4
NCHUNK = TOK // T
NBE = 3
NBR = 2
NU = 16
IST = 8
EPS = 1e-5


def _allsum16(x):
    iota = lax.iota(jnp.int32, L)
    dnums = lax.GatherDimensionNumbers(
        offset_dims=(), collapsed_slice_dims=(0,), start_index_map=(0,))
    for k in (1, 2, 4, 8):
        perm = lax.bitwise_and(iota + jnp.int32(k), jnp.int32(L - 1))
        x = x + lax.gather(
            x, perm[:, None], dnums, slice_sizes=(1,),
            unique_indices=True, indices_are_sorted=False,
            mode=lax.GatherScatterMode.PROMISE_IN_BOUNDS)
    return x


def _rsqrt16(v):
    i = lax.bitcast_convert_type(v, jnp.int32)
    i = jnp.int32(0x5F3759DF) - lax.shift_right_arithmetic(i, jnp.int32(1))
    y = lax.bitcast_convert_type(i, jnp.float32)
    half = jnp.float32(0.5) * v
    for _ in range(3):
        y = y * (jnp.float32(1.5) - half * y * y)
    return y


def _sc_body(emb_hbm, idx_hbm, table_hbm, gamma_hbm, beta_hbm, out_hbm,
             idx_v, bufe_v, bufr_v, gamma_v, beta_v,
             sem_emb, sem_gad, sem_out):
    wid = lax.axis_index("s") * NC + lax.axis_index("c")
    base = wid * TOK

    pltpu.sync_copy(idx_hbm.at[pl.ds(wid * (NCHUNK * IST), NCHUNK * IST)],
                    idx_v)
    pltpu.sync_copy(gamma_hbm, gamma_v)
    pltpu.sync_copy(beta_hbm, beta_v)

    def _addone(i, c):
        sl = pl.ds(i * L, L)
        idx_v[sl] = idx_v[sl] + jnp.int32(1)
        return c
    lax.fori_loop(0, NCHUNK * IST // L, _addone, 0)

    inv_h = jnp.float32(1.0 / H)
    zero = jnp.zeros((L,), jnp.float32)

    def emb_start(ct):
        pltpu.async_copy(emb_hbm.at[pl.ds(base + ct * T, T)],
                         bufe_v.at[lax.rem(ct, NBE)], sem_emb)

    def emb_wait():
        pltpu.make_async_copy(emb_hbm.at[pl.ds(base, T)],
                              bufe_v.at[0], sem_emb).wait()

    def gad_start(ct):
        pltpu.async_copy(table_hbm.at[idx_v.at[pl.ds(ct * IST, T)]],
                         bufr_v.at[lax.rem(ct, NBR)], sem_gad)

    def gad_wait():
        pltpu.make_async_copy(emb_hbm.at[pl.ds(base, T)],
                              bufr_v.at[0], sem_gad).wait()

    def out_start(ct):
        pltpu.async_copy(bufe_v.at[lax.rem(ct, NBE)],
                         out_hbm.at[pl.ds(base + ct * T, T)], sem_out)

    def out_wait():
        pltpu.make_async_copy(bufe_v.at[0],
                              out_hbm.at[pl.ds(base, T)], sem_out).wait()

    def compute(ct):
        be = lax.rem(ct, NBE)
        br = lax.rem(ct, NBR)
        aa = []
        cc = []
        for t in range(T):
            def _p1(j, carry):
                s = list(carry[:4])
                q = list(carry[4:])
                for k in range(8):
                    sl = pl.ds((j + k) * L, L)
                    x = bufe_v[be, t, sl] + bufr_v[br, t, sl]
                    bufe_v[be, t, sl] = x
                    s[k % 4] = s[k % 4] + x
                    q[k % 4] = q[k % 4] + x * x
                return (*s, *q)

            r = plsc.parallel_loop(0, H // L, step=8, unroll=2,
                                   carry=(zero,) * 8)(_p1)
            s = (r[0] + r[1]) + (r[2] + r[3])
            q = (r[4] + r[5]) + (r[6] + r[7])
            mean = _allsum16(s) * inv_h
            var = _allsum16(q) * inv_h - mean * mean
            rstd = _rsqrt16(var + jnp.float32(EPS))
            aa.append(rstd)
            cc.append(-mean * rstd)

        def _p2(j):
            for k in range(2):
                sl = pl.ds((j + k) * L, L)
                g = gamma_v[sl]
                bt = beta_v[sl]
                for t in range(T):
                    x = bufe_v[be, t, sl]
                    bufe_v[be, t, sl] = (x * aa[t] + cc[t]) * g + bt

        plsc.parallel_loop(0, H // L, step=2, unroll=2)(_p2)

    emb_start(0)
    gad_start(0)
    emb_start(1)
    gad_start(1)

    def _chunk(ct, c):
        @pl.when(ct + 2 < NCHUNK)
        def _():
            @pl.when(ct >= 1)
            def _():
                out_wait()
            emb_start(ct + 2)

        emb_wait()
        gad_wait()
        compute(ct)
        out_start(ct)

        @pl.when(ct + 2 < NCHUNK)
        def _():
            gad_start(ct + 2)
        return c

    lax.fori_loop(0, NCHUNK, _chunk, 0)
    out_wait()
    out_wait()
    out_wait()


@jax.jit
def _run(emb2d, idx1d, table, gamma, beta):
    mesh = plsc.VectorSubcoreMesh(core_axis_name="c", subcore_axis_name="s",
                                  num_cores=NC, num_subcores=NS)
    f = pl.kernel(
        _sc_body,
        out_type=jax.ShapeDtypeStruct((N, H), jnp.float32),
        mesh=mesh,
        scratch_types=[
            pltpu.VMEM((TOK // T * IST,), jnp.int32),
            pltpu.VMEM((NBE, T, H), jnp.float32),
            pltpu.VMEM((NBR, T, H), jnp.float32),
            pltpu.VMEM((H,), jnp.float32),
            pltpu.VMEM((H,), jnp.float32),
            pltpu.SemaphoreType.DMA,
            pltpu.SemaphoreType.DMA,
            pltpu.SemaphoreType.DMA,
        ],
    )
    return f(emb2d, idx1d, table, gamma, beta)


def kernel(embedded_input_ids, position_ids, pos_table, ln_gamma, ln_beta):
    emb2d = embedded_input_ids.reshape(N, H)
    idx1d = position_ids.astype(jnp.int32).reshape(N)
    idxp = (jnp.zeros((N // T, IST), jnp.int32)
            .at[:, :T].set(idx1d.reshape(N // T, T)).reshape(-1))
    out = _run(emb2d, idxp, pos_table, ln_gamma, ln_beta)
    return out.reshape(B, S, H)


# speedup vs baseline: 2.4799x; 1.0209x over previous
import jax
import jax.numpy as jnp
from jax import lax
from jax.experimental import pallas as pl
from jax.experimental.pallas import tpu as pltpu
from jax.experimental.pallas import tpu_sc as plsc

B, S, H = 4, 2048, 4096
N = B * S
NC, NS, L = 2, 16, 16
NW = NC * NS
TOK = N // NW
T = 4
NCHUNK = TOK // T
NBE = 3
NBR = 2
NU = 16
IST = 8
EPS = 1e-5


def _allsum16(x):
    iota = lax.iota(jnp.int32, L)
    dnums = lax.GatherDimensionNumbers(
        offset_dims=(), collapsed_slice_dims=(0,), start_index_map=(0,))
    for k in (1, 2, 4, 8):
        perm = lax.bitwise_and(iota + jnp.int32(k), jnp.int32(L - 1))
        x = x + lax.gather(
            x, perm[:, None], dnums, slice_sizes=(1,),
            unique_indices=True, indices_are_sorted=False,
            mode=lax.GatherScatterMode.PROMISE_IN_BOUNDS)
    return x


def _rsqrt16(v):
    i = lax.bitcast_convert_type(v, jnp.int32)
    i = jnp.int32(0x5F3759DF) - lax.shift_right_arithmetic(i, jnp.int32(1))
    y = lax.bitcast_convert_type(i, jnp.float32)
    half = jnp.float32(0.5) * v
    for _ in range(3):
        y = y * (jnp.float32(1.5) - half * y * y)
    return y


def _sc_body(emb_hbm, idx_hbm, table_hbm, gamma_hbm, beta_hbm, out_hbm,
             idx_v, bufe_v, bufr_v, gamma_v, beta_v,
             sem_emb, sem_gad, sem_out):
    wid = lax.axis_index("s") * NC + lax.axis_index("c")
    base = wid * TOK

    pltpu.sync_copy(idx_hbm.at[pl.ds(wid * (NCHUNK * IST), NCHUNK * IST)],
                    idx_v)
    pltpu.sync_copy(gamma_hbm, gamma_v)
    pltpu.sync_copy(beta_hbm, beta_v)

    def _addone(i, c):
        sl = pl.ds(i * L, L)
        idx_v[sl] = idx_v[sl] + jnp.int32(1)
        return c
    lax.fori_loop(0, NCHUNK * IST // L, _addone, 0)

    inv_h = jnp.float32(1.0 / H)
    zero = jnp.zeros((L,), jnp.float32)

    def emb_start(ct):
        pltpu.async_copy(emb_hbm.at[pl.ds(base + ct * T, T)],
                         bufe_v.at[lax.rem(ct, NBE)], sem_emb)

    def emb_wait():
        pltpu.make_async_copy(emb_hbm.at[pl.ds(base, T)],
                              bufe_v.at[0], sem_emb).wait()

    def gad_start(ct):
        pltpu.async_copy(table_hbm.at[idx_v.at[pl.ds(ct * IST, T)]],
                         bufr_v.at[lax.rem(ct, NBR)], sem_gad)

    def gad_wait():
        pltpu.make_async_copy(emb_hbm.at[pl.ds(base, T)],
                              bufr_v.at[0], sem_gad).wait()

    def out_start(ct):
        pltpu.async_copy(bufe_v.at[lax.rem(ct, NBE)],
                         out_hbm.at[pl.ds(base + ct * T, T)], sem_out)

    def out_wait():
        pltpu.make_async_copy(bufe_v.at[0],
                              out_hbm.at[pl.ds(base, T)], sem_out).wait()

    def compute(ct):
        be = lax.rem(ct, NBE)
        br = lax.rem(ct, NBR)
        aa = []
        cc = []

        def _p1(j, carry):
            s = list(carry[:T])
            q = list(carry[T:])
            for k in range(2):
                for t in range(T):
                    sl = pl.ds((j + k) * L, L)
                    x = bufe_v[be, t, sl] + bufr_v[br, t, sl]
                    bufe_v[be, t, sl] = x
                    s[t] = s[t] + x
                    q[t] = q[t] + x * x
            return (*s, *q)

        r = plsc.parallel_loop(0, H // L, step=2, unroll=2,
                               carry=(zero,) * (2 * T))(_p1)
        for t in range(T):
            mean = _allsum16(r[t]) * inv_h
            var = _allsum16(r[T + t]) * inv_h - mean * mean
            rstd = _rsqrt16(var + jnp.float32(EPS))
            aa.append(rstd)
            cc.append(-mean * rstd)

        def _p2(j):
            for k in range(2):
                sl = pl.ds((j + k) * L, L)
                g = gamma_v[sl]
                bt = beta_v[sl]
                for t in range(T):
                    x = bufe_v[be, t, sl]
                    bufe_v[be, t, sl] = (x * aa[t] + cc[t]) * g + bt

        plsc.parallel_loop(0, H // L, step=2, unroll=2)(_p2)

    emb_start(0)
    gad_start(0)
    emb_start(1)
    gad_start(1)

    def _chunk(ct, c):
        @pl.when(ct + 2 < NCHUNK)
        def _():
            @pl.when(ct >= 1)
            def _():
                out_wait()
            emb_start(ct + 2)

        emb_wait()
        gad_wait()
        compute(ct)
        out_start(ct)

        @pl.when(ct + 2 < NCHUNK)
        def _():
            gad_start(ct + 2)
        return c

    lax.fori_loop(0, NCHUNK, _chunk, 0)
    out_wait()
    out_wait()
    out_wait()


@jax.jit
def _run(emb2d, idx1d, table, gamma, beta):
    mesh = plsc.VectorSubcoreMesh(core_axis_name="c", subcore_axis_name="s",
                                  num_cores=NC, num_subcores=NS)
    f = pl.kernel(
        _sc_body,
        out_type=jax.ShapeDtypeStruct((N, H), jnp.float32),
        mesh=mesh,
        scratch_types=[
            pltpu.VMEM((TOK // T * IST,), jnp.int32),
            pltpu.VMEM((NBE, T, H), jnp.float32),
            pltpu.VMEM((NBR, T, H), jnp.float32),
            pltpu.VMEM((H,), jnp.float32),
            pltpu.VMEM((H,), jnp.float32),
            pltpu.SemaphoreType.DMA,
            pltpu.SemaphoreType.DMA,
            pltpu.SemaphoreType.DMA,
        ],
    )
    return f(emb2d, idx1d, table, gamma, beta)


def kernel(embedded_input_ids, position_ids, pos_table, ln_gamma, ln_beta):
    emb2d = embedded_input_ids.reshape(N, H)
    idx1d = position_ids.astype(jnp.int32).reshape(N)
    idxp = (jnp.zeros((N // T, IST), jnp.int32)
            .at[:, :T].set(idx1d.reshape(N // T, T)).reshape(-1))
    out = _run(emb2d, idxp, pos_table, ln_gamma, ln_beta)
    return out.reshape(B, S, H)


# speedup vs baseline: 6.3595x; 2.5644x over previous
import jax
import jax.numpy as jnp
from jax import lax
from jax.experimental import pallas as pl
from jax.experimental.pallas import tpu as pltpu
from jax.experimental.pallas import tpu_sc as plsc

B, S, H = 4, 2048, 4096
N = B * S
NC, NS, L = 2, 16, 16
NW = NC * NS
TOK = N // NW
T = 4
NCHUNK = TOK // T
NBE = 3
NBR = 3
NU = 16
IST = 8
EPS = 1e-5


def _allsum16(x):
    iota = lax.iota(jnp.int32, L)
    dnums = lax.GatherDimensionNumbers(
        offset_dims=(), collapsed_slice_dims=(0,), start_index_map=(0,))
    for k in (1, 2, 4, 8):
        perm = lax.bitwise_and(iota + jnp.int32(k), jnp.int32(L - 1))
        x = x + lax.gather(
            x, perm[:, None], dnums, slice_sizes=(1,),
            unique_indices=True, indices_are_sorted=False,
            mode=lax.GatherScatterMode.PROMISE_IN_BOUNDS)
    return x


def _rsqrt16(v):
    i = lax.bitcast_convert_type(v, jnp.int32)
    i = jnp.int32(0x5F3759DF) - lax.shift_right_arithmetic(i, jnp.int32(1))
    y = lax.bitcast_convert_type(i, jnp.float32)
    half = jnp.float32(0.5) * v
    for _ in range(3):
        y = y * (jnp.float32(1.5) - half * y * y)
    return y


def _sc_body(emb_hbm, idx_hbm, table_hbm, gamma_hbm, beta_hbm, out_hbm,
             idx_v, bufe_v, bufr_v, gamma_v, beta_v,
             sem_emb, sem_gad, sem_out):
    wid = lax.axis_index("s") * NC + lax.axis_index("c")
    base = wid * TOK

    pltpu.sync_copy(idx_hbm.at[pl.ds(wid * (NCHUNK * IST), NCHUNK * IST)],
                    idx_v)
    pltpu.sync_copy(gamma_hbm, gamma_v)
    pltpu.sync_copy(beta_hbm, beta_v)

    def _addone(i, c):
        sl = pl.ds(i * L, L)
        idx_v[sl] = idx_v[sl] + jnp.int32(1)
        return c
    lax.fori_loop(0, NCHUNK * IST // L, _addone, 0)

    inv_h = jnp.float32(1.0 / H)
    zero = jnp.zeros((L,), jnp.float32)

    def emb_start(ct):
        pltpu.async_copy(emb_hbm.at[pl.ds(base + ct * T, T)],
                         bufe_v.at[lax.rem(ct, NBE)], sem_emb)

    def emb_wait():
        pltpu.make_async_copy(emb_hbm.at[pl.ds(base, T)],
                              bufe_v.at[0], sem_emb).wait()

    def gad_start(ct):
        pltpu.async_copy(table_hbm.at[idx_v.at[pl.ds(ct * IST, T)]],
                         bufr_v.at[lax.rem(ct, NBR)], sem_gad)

    def gad_wait():
        pltpu.make_async_copy(emb_hbm.at[pl.ds(base, T)],
                              bufr_v.at[0], sem_gad).wait()

    def out_start(ct):
        pltpu.async_copy(bufe_v.at[lax.rem(ct, NBE)],
                         out_hbm.at[pl.ds(base + ct * T, T)], sem_out)

    def out_wait():
        pltpu.make_async_copy(bufe_v.at[0],
                              out_hbm.at[pl.ds(base, T)], sem_out).wait()

    def compute(ct):
        be = lax.rem(ct, NBE)
        br = lax.rem(ct, NBR)
        aa = []
        cc = []

        def _p1(j, carry):
            s = list(carry[:T])
            q = list(carry[T:])
            for k in range(2):
                for t in range(T):
                    sl = pl.ds((j + k) * L, L)
                    x = bufe_v[be, t, sl] + bufr_v[br, t, sl]
                    bufe_v[be, t, sl] = x
                    s[t] = s[t] + x
                    q[t] = q[t] + x * x
            return (*s, *q)

        r = plsc.parallel_loop(0, H // L, step=2, unroll=2,
                               carry=(zero,) * (2 * T))(_p1)
        for t in range(T):
            mean = _allsum16(r[t]) * inv_h
            var = _allsum16(r[T + t]) * inv_h - mean * mean
            rstd = _rsqrt16(var + jnp.float32(EPS))
            aa.append(rstd)
            cc.append(-mean * rstd)

        def _p2(j):
            for k in range(2):
                sl = pl.ds((j + k) * L, L)
                g = gamma_v[sl]
                bt = beta_v[sl]
                for t in range(T):
                    x = bufe_v[be, t, sl]
                    bufe_v[be, t, sl] = (x * aa[t] + cc[t]) * g + bt

        plsc.parallel_loop(0, H // L, step=2, unroll=2)(_p2)

    emb_start(0)
    gad_start(0)
    emb_start(1)
    gad_start(1)

    def _chunk(ct, c):
        @pl.when(ct + 2 < NCHUNK)
        def _():
            @pl.when(ct >= 1)
            def _():
                out_wait()
            emb_start(ct + 2)
            gad_start(ct + 2)

        emb_wait()
        gad_wait()
        compute(ct)
        out_start(ct)
        return c

    lax.fori_loop(0, NCHUNK, _chunk, 0)
    out_wait()
    out_wait()
    out_wait()


@jax.jit
def _run(emb2d, idx1d, table, gamma, beta):
    mesh = plsc.VectorSubcoreMesh(core_axis_name="c", subcore_axis_name="s",
                                  num_cores=NC, num_subcores=NS)
    f = pl.kernel(
        _sc_body,
        out_type=jax.ShapeDtypeStruct((N, H), jnp.float32),
        mesh=mesh,
        scratch_types=[
            pltpu.VMEM((TOK // T * IST,), jnp.int32),
            pltpu.VMEM((NBE, T, H), jnp.float32),
            pltpu.VMEM((NBR, T, H), jnp.float32),
            pltpu.VMEM((H,), jnp.float32),
            pltpu.VMEM((H,), jnp.float32),
            pltpu.SemaphoreType.DMA,
            pltpu.SemaphoreType.DMA,
            pltpu.SemaphoreType.DMA,
        ],
    )
    return f(emb2d, idx1d, table, gamma, beta)


def kernel(embedded_input_ids, position_ids, pos_table, ln_gamma, ln_beta):
    emb2d = embedded_input_ids.reshape(N, H)
    idx1d = position_ids.astype(jnp.int32).reshape(N)
    idxp = (jnp.zeros((N // T, IST), jnp.int32)
            .at[:, :T].set(idx1d.reshape(N // T, T)).reshape(-1))
    out = _run(emb2d, idxp, pos_table, ln_gamma, ln_beta)
    return out.reshape(B, S, H)
